# Initial kernel scaffold; baseline (speedup 1.0000x reference)
import jax
import jax.numpy as jnp
from jax.experimental import pallas as pl


def kernel(inputs, left_idx, right_idx, W_gih, b_gih, W_glhh, W_grhh, W_cih, b_cih, W_clhh, W_crhh):
    raise NotImplementedError("write your pallas kernel here")


# speedup vs baseline: 5.7720x; 5.7720x over previous
import functools

import jax
import jax.numpy as jnp
from jax.experimental import pallas as pl
from jax.experimental.pallas import tpu as pltpu


def _proj_kernel(x_ref, w_ref, b_ref, o_ref):
    o_ref[:, :] = (
        jnp.dot(x_ref[:, :], w_ref[:, :], preferred_element_type=jnp.float32)
        + b_ref[:, :]
    )


def _scan_kernel(xb_ref, lf_ref, rf_ref, wg_ref, wc_ref, out_ref, h_scr, lrh_scr,
                 *, L, B, H):
    h_scr[pl.ds(L * B, B), :] = jnp.zeros((B, H), jnp.float32)

    def body(t, carry):
        for b in range(B):
            li = lf_ref[t, b]
            ri = rf_ref[t, b]
            lrh_scr[pl.ds(b, 1), 0:H] = h_scr[pl.ds(li, 1), :]
            lrh_scr[pl.ds(b, 1), H:2 * H] = h_scr[pl.ds(ri, 1), :]
        lrh = lrh_scr[:, :]
        lh = lrh[:, 0:H]
        rh = lrh[:, H:2 * H]
        xbt = xb_ref[t]
        gates = jax.nn.sigmoid(
            xbt[:, 0:5 * H]
            + jnp.dot(lrh, wg_ref[:, :], preferred_element_type=jnp.float32)
        )
        rl = gates[:, 0:H]
        rr = gates[:, H:2 * H]
        zl = gates[:, 2 * H:3 * H]
        zr = gates[:, 3 * H:4 * H]
        z = gates[:, 4 * H:5 * H]
        lrh_scr[:, 0:H] = rl * lh
        lrh_scr[:, H:2 * H] = rr * rh
        cell = jnp.tanh(
            xbt[:, 5 * H:6 * H]
            + jnp.dot(lrh_scr[:, :], wc_ref[:, :],
                      preferred_element_type=jnp.float32)
        )
        h = zl * lh + zr * rh + z * cell
        h_scr[pl.ds(t * B, B), :] = h
        out_ref[t] = h
        return carry

    jax.lax.fori_loop(0, L, body, 0)


def kernel(inputs, left_idx, right_idx, W_gih, b_gih, W_glhh, W_grhh,
           W_cih, b_cih, W_clhh, W_crhh):
    L, B, D = inputs.shape
    H = W_cih.shape[0]

    Wx = jnp.concatenate([W_gih, W_cih], axis=0).T
    bx = jnp.concatenate([b_gih, b_cih], axis=0)[None, :]
    Wg = jnp.concatenate([W_glhh, W_grhh], axis=1).T
    Wc = jnp.concatenate([W_clhh, W_crhh], axis=1).T

    tvec = jnp.arange(L, dtype=jnp.int32)[:, None]
    bvec = jnp.arange(B, dtype=jnp.int32)[None, :]
    lf = jnp.where(left_idx < tvec,
                   jnp.clip(left_idx, 0, L - 1) * B + bvec, L * B + bvec)
    rf = jnp.where(right_idx < tvec,
                   jnp.clip(right_idx, 0, L - 1) * B + bvec, L * B + bvec)

    x_flat = inputs.reshape(L * B, D)
    TM = 256
    xb = pl.pallas_call(
        _proj_kernel,
        grid=(L * B // TM,),
        in_specs=[
            pl.BlockSpec((TM, D), lambda i: (i, 0)),
            pl.BlockSpec((D, 6 * H), lambda i: (0, 0)),
            pl.BlockSpec((1, 6 * H), lambda i: (0, 0)),
        ],
        out_specs=pl.BlockSpec((TM, 6 * H), lambda i: (i, 0)),
        out_shape=jax.ShapeDtypeStruct((L * B, 6 * H), jnp.float32),
    )(x_flat, Wx, bx)
    xb = xb.reshape(L, B, 6 * H)

    hs = pl.pallas_call(
        functools.partial(_scan_kernel, L=L, B=B, H=H),
        in_specs=[
            pl.BlockSpec(memory_space=pltpu.VMEM),
            pl.BlockSpec(memory_space=pltpu.SMEM),
            pl.BlockSpec(memory_space=pltpu.SMEM),
            pl.BlockSpec(memory_space=pltpu.VMEM),
            pl.BlockSpec(memory_space=pltpu.VMEM),
        ],
        out_specs=pl.BlockSpec(memory_space=pltpu.VMEM),
        out_shape=jax.ShapeDtypeStruct((L, B, H), jnp.float32),
        scratch_shapes=[
            pltpu.VMEM((L * B + B, H), jnp.float32),
            pltpu.VMEM((B, 2 * H), jnp.float32),
        ],
    )(xb, lf, rf, Wg, Wc)

    return jnp.transpose(hs, (1, 0, 2))


# speedup vs baseline: 6.5864x; 1.1411x over previous
import functools

import jax
import jax.numpy as jnp
from jax.experimental import pallas as pl
from jax.experimental.pallas import tpu as pltpu


def _fused_kernel(x_ref, wx_ref, bx_ref, lf_ref, rf_ref, wg_ref, wc_ref,
                  out_ref, xb_scr, h_scr, lrh_scr, *, L, B, H):
    xb_scr[:, :] = (
        jnp.dot(x_ref[:, :], wx_ref[:, :], preferred_element_type=jnp.float32)
        + bx_ref[:, :]
    )
    h_scr[pl.ds(L * B, B), :] = jnp.zeros((B, H), jnp.float32)

    def body(t, carry):
        for b in range(B):
            li = lf_ref[t, b]
            ri = rf_ref[t, b]
            lrh_scr[pl.ds(b, 1), 0:H] = h_scr[pl.ds(li, 1), :]
            lrh_scr[pl.ds(b, 1), H:2 * H] = h_scr[pl.ds(ri, 1), :]
        lrh = lrh_scr[:, :]
        lh = lrh[:, 0:H]
        rh = lrh[:, H:2 * H]
        xbt = xb_scr[pl.ds(t * B, B), :]
        gates = jax.nn.sigmoid(
            xbt[:, 0:5 * H]
            + jnp.dot(lrh, wg_ref[:, :], preferred_element_type=jnp.float32)
        )
        rl = gates[:, 0:H]
        rr = gates[:, H:2 * H]
        zl = gates[:, 2 * H:3 * H]
        zr = gates[:, 3 * H:4 * H]
        z = gates[:, 4 * H:5 * H]
        lrh_scr[:, 0:H] = rl * lh
        lrh_scr[:, H:2 * H] = rr * rh
        cell = jnp.tanh(
            xbt[:, 5 * H:6 * H]
            + jnp.dot(lrh_scr[:, :], wc_ref[:, :],
                      preferred_element_type=jnp.float32)
        )
        h = zl * lh + zr * rh + z * cell
        h_scr[pl.ds(t * B, B), :] = h
        out_ref[t] = h
        return carry

    jax.lax.fori_loop(0, L, body, 0)


def kernel(inputs, left_idx, right_idx, W_gih, b_gih, W_glhh, W_grhh,
           W_cih, b_cih, W_clhh, W_crhh):
    L, B, D = inputs.shape
    H = W_cih.shape[0]

    x_flat = inputs.reshape(L * B, D)
    Wx = jnp.concatenate([W_gih, W_cih], axis=0).T
    bx = jnp.concatenate([b_gih, b_cih], axis=0)[None, :]
    Wg = jnp.concatenate([W_glhh, W_grhh], axis=1).T
    Wc = jnp.concatenate([W_clhh, W_crhh], axis=1).T

    tvec = jnp.arange(L, dtype=jnp.int32)[:, None]
    bvec = jnp.arange(B, dtype=jnp.int32)[None, :]
    lf = jnp.where(left_idx < tvec,
                   jnp.clip(left_idx, 0, L - 1) * B + bvec, L * B + bvec)
    rf = jnp.where(right_idx < tvec,
                   jnp.clip(right_idx, 0, L - 1) * B + bvec, L * B + bvec)

    hs = pl.pallas_call(
        functools.partial(_fused_kernel, L=L, B=B, H=H),
        in_specs=[
            pl.BlockSpec(memory_space=pltpu.VMEM),
            pl.BlockSpec(memory_space=pltpu.VMEM),
            pl.BlockSpec(memory_space=pltpu.VMEM),
            pl.BlockSpec(memory_space=pltpu.SMEM),
            pl.BlockSpec(memory_space=pltpu.SMEM),
            pl.BlockSpec(memory_space=pltpu.VMEM),
            pl.BlockSpec(memory_space=pltpu.VMEM),
        ],
        out_specs=pl.BlockSpec(memory_space=pltpu.VMEM),
        out_shape=jax.ShapeDtypeStruct((L, B, H), jnp.float32),
        scratch_shapes=[
            pltpu.VMEM((L * B, 6 * H), jnp.float32),
            pltpu.VMEM((L * B + B, H), jnp.float32),
            pltpu.VMEM((B, 2 * H), jnp.float32),
        ],
    )(x_flat, Wx, bx, lf, rf, Wg, Wc)

    return jnp.transpose(hs, (1, 0, 2))
